# Initial kernel scaffold; baseline (speedup 1.0000x reference)
import jax
import jax.numpy as jnp
from jax.experimental import pallas as pl


def kernel(x, edge_index, batch, W1, b1, W2, b2, fc_w, fc_b):
    raise NotImplementedError("write your pallas kernel here")


# speedup vs baseline: 17.1252x; 17.1252x over previous
import functools

import jax
import jax.numpy as jnp
from jax import lax
from jax.experimental import pallas as pl
from jax.experimental.pallas import tpu as pltpu
from jax.experimental.pallas import tpu_sc as plsc

N = 10000
E = 320000
F_IN = 128
H1 = 64
H2 = 32
C = 3
G = 64

NCORE = 2
NSUB = 16
NW = NCORE * NSUB
K = 128
EPW = 10240
CH = EPW // K
NP = 10240
RPT = NP // NSUB


def _sc_mesh():
    return plsc.VectorSubcoreMesh(core_axis_name="c", subcore_axis_name="s")


def _zero_rows_buf(rows, h):
    def body(r, carry):
        for i in range(h // 16):
            rows[r, pl.ds(i * 16, 16)] = jnp.zeros((16,), jnp.float32)
        return carry
    lax.fori_loop(0, K, body, 0)


def _zero_acc_slice(rows, acc, base):
    nfull = RPT // K
    rem = RPT % K
    for b in range(nfull):
        pltpu.sync_copy(rows, acc.at[pl.ds(base + b * K, K)])
    if rem:
        pltpu.sync_copy(rows.at[pl.ds(0, rem)],
                        acc.at[pl.ds(base + nfull * K, rem)])


def _make_deg():
    @functools.partial(
        pl.kernel,
        out_type=jax.ShapeDtypeStruct((NCORE, NP, 16), jnp.float32),
        mesh=_sc_mesh(),
        scratch_types=[
            pltpu.VMEM((CH, K), jnp.int32),
            pltpu.VMEM((K, 16), jnp.float32),
            pltpu.VMEM_SHARED((NP, 16), jnp.float32),
        ],
    )
    def deg(dst_hbm, out_hbm, didx, rows, acc):
        c = lax.axis_index("c")
        s = lax.axis_index("s")
        w = s * NCORE + c
        base = s * RPT
        _zero_rows_buf(rows, 16)
        _zero_acc_slice(rows, acc, base)
        pltpu.sync_copy(dst_hbm.at[w], didx)

        def fill_ones(r, carry):
            rows[r, :] = jnp.ones((16,), jnp.float32)
            return carry
        lax.fori_loop(0, K, fill_ones, 0)
        plsc.subcore_barrier()

        def chunk(j, carry):
            pltpu.sync_copy(rows, acc.at[didx.at[j]], add=True)
            return carry
        lax.fori_loop(0, CH, chunk, 0)
        plsc.subcore_barrier()
        pltpu.sync_copy(acc.at[pl.ds(base, RPT)],
                        out_hbm.at[c, pl.ds(base, RPT)])

    return deg


def _make_agg(h):
    @functools.partial(
        pl.kernel,
        out_type=jax.ShapeDtypeStruct((NCORE, NP, h), jnp.float32),
        mesh=_sc_mesh(),
        scratch_types=[
            pltpu.VMEM((CH, K), jnp.int32),
            pltpu.VMEM((CH, K), jnp.int32),
            pltpu.VMEM((K, h), jnp.float32),
            pltpu.VMEM_SHARED((NP, h), jnp.float32),
            pltpu.SemaphoreType.DMA,
        ],
        compiler_params=pltpu.CompilerParams(use_tc_tiling_on_sc=False),
    )
    def agg(table_hbm, src_hbm, dst_hbm, out_hbm, sidx, didx, rows, acc, sem):
        c = lax.axis_index("c")
        s = lax.axis_index("s")
        w = s * NCORE + c
        base = s * RPT
        _zero_rows_buf(rows, h)
        _zero_acc_slice(rows, acc, base)
        pltpu.sync_copy(src_hbm.at[w], sidx)
        pltpu.sync_copy(dst_hbm.at[w], didx)
        plsc.subcore_barrier()

        def chunk(j, carry):
            pltpu.async_copy(table_hbm.at[sidx.at[j]], rows, sem).wait()
            pltpu.sync_copy(rows, acc.at[didx.at[j]], add=True)
            return carry
        lax.fori_loop(0, CH, chunk, 0)
        plsc.subcore_barrier()
        pltpu.sync_copy(acc.at[pl.ds(base, RPT)],
                        out_hbm.at[c, pl.ds(base, RPT)])

    return agg


def _mm1(x, w1):
    def body(x_ref, w_ref, o_ref):
        o_ref[...] = jnp.dot(x_ref[...], w_ref[...],
                             preferred_element_type=jnp.float32)
    return pl.pallas_call(
        body, out_shape=jax.ShapeDtypeStruct((N, H1), jnp.float32))(x, w1)


def _scale(hw1, deg16):
    def body(hw_ref, deg_ref, hwp_ref, dinv_ref):
        deg = deg_ref[0, :N, 0] + deg_ref[1, :N, 0] + 1.0
        dinv = lax.rsqrt(deg)[:, None]
        dinv_ref[...] = dinv
        hwp_ref[...] = hw_ref[...] * dinv
    return pl.pallas_call(
        body,
        out_shape=[jax.ShapeDtypeStruct((N, H1), jnp.float32),
                   jax.ShapeDtypeStruct((N, 1), jnp.float32)])(hw1, deg16)


def _layer2(agg1, hw1p, dinv, b1, w2):
    def body(agg_ref, hwp_ref, dinv_ref, b_ref, w_ref, o_ref):
        aggsum = agg_ref[0, :N, :] + agg_ref[1, :N, :] + hwp_ref[...]
        hcur = jnp.maximum(aggsum * dinv_ref[...] + b_ref[...], 0.0)
        hw2 = jnp.dot(hcur, w_ref[...], preferred_element_type=jnp.float32)
        o_ref[...] = hw2 * dinv_ref[...]
    return pl.pallas_call(
        body, out_shape=jax.ShapeDtypeStruct((N, H2), jnp.float32))(
            agg1, hw1p, dinv, b1, w2)


def _final(agg2, hw2p, dinv, b2, batch2d, fc_w, fc_b):
    def body(agg_ref, hwp_ref, dinv_ref, b_ref, bat_ref, fw_ref, fb_ref,
             o_ref):
        aggsum = agg_ref[0, :N, :] + agg_ref[1, :N, :] + hwp_ref[...]
        hcur = jnp.maximum(aggsum * dinv_ref[...] + b_ref[...], 0.0)
        onehot = (bat_ref[...] == lax.broadcasted_iota(
            jnp.int32, (N, G), 1)).astype(jnp.float32)
        sums = lax.dot_general(onehot, hcur, (((0,), (0,)), ((), ())),
                               preferred_element_type=jnp.float32)
        cnt = jnp.sum(onehot, axis=0)[:, None]
        pooled = sums / jnp.maximum(cnt, 1.0)
        o_ref[...] = jnp.dot(pooled, fw_ref[...],
                             preferred_element_type=jnp.float32) + fb_ref[...]
    return pl.pallas_call(
        body, out_shape=jax.ShapeDtypeStruct((G, C), jnp.float32))(
            agg2, hw2p, dinv, b2, batch2d, fc_w, fc_b)


def kernel(x, edge_index, batch, W1, b1, W2, b2, fc_w, fc_b):
    pad = NW * EPW - E
    src_r = jnp.concatenate(
        [edge_index[0], jnp.zeros((pad,), jnp.int32)]).reshape(NW, CH, K)
    dst_r = jnp.concatenate(
        [edge_index[1], jnp.full((pad,), N, jnp.int32)]).reshape(NW, CH, K)

    deg16 = _make_deg()(dst_r)
    hw1 = _mm1(x, W1)
    hw1p, dinv = _scale(hw1, deg16)
    agg1 = _make_agg(H1)(hw1p, src_r, dst_r)
    hw2p = _layer2(agg1, hw1p, dinv, b1.reshape(1, H1), W2)
    agg2 = _make_agg(H2)(hw2p, src_r, dst_r)
    return _final(agg2, hw2p, dinv, b2.reshape(1, H2),
                  batch.reshape(N, 1), fc_w, fc_b)


# speedup vs baseline: 18.6889x; 1.0913x over previous
import functools

import jax
import jax.numpy as jnp
from jax import lax
from jax.experimental import pallas as pl
from jax.experimental.pallas import tpu as pltpu
from jax.experimental.pallas import tpu_sc as plsc

N = 10000
E = 320000
F_IN = 128
H1 = 64
H2 = 32
C = 3
G = 64

NCORE = 2
NSUB = 16
NW = NCORE * NSUB
K = 128
EPW = 10240
CH = EPW // K
NP = 10240
RPT = NP // NSUB


def _sc_mesh():
    return plsc.VectorSubcoreMesh(core_axis_name="c", subcore_axis_name="s")


def _zero_rows_buf(rows, h):
    def body(r, carry):
        for i in range(h // 16):
            rows[r, pl.ds(i * 16, 16)] = jnp.zeros((16,), jnp.float32)
        return carry
    lax.fori_loop(0, K, body, 0)


def _zero_acc_slice(rows, acc, base):
    nfull = RPT // K
    rem = RPT % K
    for b in range(nfull):
        pltpu.sync_copy(rows, acc.at[pl.ds(base + b * K, K)])
    if rem:
        pltpu.sync_copy(rows.at[pl.ds(0, rem)],
                        acc.at[pl.ds(base + nfull * K, rem)])


def _make_deg():
    @functools.partial(
        pl.kernel,
        out_type=jax.ShapeDtypeStruct((NCORE, NP, 16), jnp.float32),
        mesh=_sc_mesh(),
        scratch_types=[
            pltpu.VMEM((CH, K), jnp.int32),
            pltpu.VMEM((K, 16), jnp.float32),
            pltpu.VMEM_SHARED((NP, 16), jnp.float32),
            pltpu.SemaphoreType.DMA,
        ],
    )
    def deg(dst_hbm, out_hbm, didx, rows, acc, sem):
        c = lax.axis_index("c")
        s = lax.axis_index("s")
        w = s * NCORE + c
        base = s * RPT
        _zero_rows_buf(rows, 16)
        _zero_acc_slice(rows, acc, base)
        pltpu.sync_copy(dst_hbm.at[w], didx)

        def fill_ones(r, carry):
            rows[r, :] = jnp.ones((16,), jnp.float32)
            return carry
        lax.fori_loop(0, K, fill_ones, 0)
        plsc.subcore_barrier()

        def chunk(j, carry):
            pltpu.async_copy(rows, acc.at[didx.at[j]], sem, add=True).wait()
            return carry
        lax.fori_loop(0, CH, chunk, 0)
        plsc.subcore_barrier()
        pltpu.sync_copy(acc.at[pl.ds(base, RPT)],
                        out_hbm.at[c, pl.ds(base, RPT)])

    return deg


NBUF = 4
ROUNDS = CH // NBUF


def _make_agg(h):
    @functools.partial(
        pl.kernel,
        out_type=jax.ShapeDtypeStruct((NCORE, NP, h), jnp.float32),
        mesh=_sc_mesh(),
        scratch_types=[
            pltpu.VMEM((CH, K), jnp.int32),
            pltpu.VMEM((CH, K), jnp.int32),
            [pltpu.VMEM((K, h), jnp.float32) for _ in range(NBUF)],
            pltpu.VMEM_SHARED((NP, h), jnp.float32),
            [pltpu.SemaphoreType.DMA for _ in range(NBUF)],
            [pltpu.SemaphoreType.DMA for _ in range(NBUF)],
        ],
        compiler_params=pltpu.CompilerParams(use_tc_tiling_on_sc=False),
    )
    def agg(table_hbm, src_hbm, dst_hbm, out_hbm, sidx, didx, rows, acc,
            sem_g, sem_s):
        c = lax.axis_index("c")
        s = lax.axis_index("s")
        w = s * NCORE + c
        base = s * RPT
        _zero_rows_buf(rows[0], h)
        _zero_acc_slice(rows[0], acc, base)
        pltpu.sync_copy(src_hbm.at[w], sidx)
        pltpu.sync_copy(dst_hbm.at[w], didx)
        plsc.subcore_barrier()

        def round_body(i, carry):
            j0 = i * NBUF
            descs = [
                pltpu.async_copy(table_hbm.at[sidx.at[j0 + b]], rows[b],
                                 sem_g[b])
                for b in range(NBUF)
            ]
            for b in range(NBUF):
                descs[b].wait()
                pltpu.async_copy(rows[b], acc.at[didx.at[j0 + b]], sem_s[b],
                                 add=True).wait()
            return carry
        lax.fori_loop(0, ROUNDS, round_body, 0)
        plsc.subcore_barrier()
        pltpu.sync_copy(acc.at[pl.ds(base, RPT)],
                        out_hbm.at[c, pl.ds(base, RPT)])

    return agg


def _mm1(x, w1):
    def body(x_ref, w_ref, o_ref):
        o_ref[...] = jnp.dot(x_ref[...], w_ref[...],
                             preferred_element_type=jnp.float32)
    return pl.pallas_call(
        body, out_shape=jax.ShapeDtypeStruct((N, H1), jnp.float32))(x, w1)


def _scale(hw1, deg16):
    def body(hw_ref, deg_ref, hwp_ref, dinv_ref):
        deg = deg_ref[0, :N, 0] + deg_ref[1, :N, 0] + 1.0
        dinv = lax.rsqrt(deg)[:, None]
        dinv_ref[...] = dinv
        hwp_ref[...] = hw_ref[...] * dinv
    return pl.pallas_call(
        body,
        out_shape=[jax.ShapeDtypeStruct((N, H1), jnp.float32),
                   jax.ShapeDtypeStruct((N, 1), jnp.float32)])(hw1, deg16)


def _layer2(agg1, hw1p, dinv, b1, w2):
    def body(agg_ref, hwp_ref, dinv_ref, b_ref, w_ref, o_ref):
        aggsum = agg_ref[0, :N, :] + agg_ref[1, :N, :] + hwp_ref[...]
        hcur = jnp.maximum(aggsum * dinv_ref[...] + b_ref[...], 0.0)
        hw2 = jnp.dot(hcur, w_ref[...], preferred_element_type=jnp.float32)
        o_ref[...] = hw2 * dinv_ref[...]
    return pl.pallas_call(
        body, out_shape=jax.ShapeDtypeStruct((N, H2), jnp.float32))(
            agg1, hw1p, dinv, b1, w2)


def _final(agg2, hw2p, dinv, b2, batch2d, fc_w, fc_b):
    def body(agg_ref, hwp_ref, dinv_ref, b_ref, bat_ref, fw_ref, fb_ref,
             o_ref):
        aggsum = agg_ref[0, :N, :] + agg_ref[1, :N, :] + hwp_ref[...]
        hcur = jnp.maximum(aggsum * dinv_ref[...] + b_ref[...], 0.0)
        onehot = (bat_ref[...] == lax.broadcasted_iota(
            jnp.int32, (N, G), 1)).astype(jnp.float32)
        sums = lax.dot_general(onehot, hcur, (((0,), (0,)), ((), ())),
                               preferred_element_type=jnp.float32)
        cnt = jnp.sum(onehot, axis=0)[:, None]
        pooled = sums / jnp.maximum(cnt, 1.0)
        o_ref[...] = jnp.dot(pooled, fw_ref[...],
                             preferred_element_type=jnp.float32) + fb_ref[...]
    return pl.pallas_call(
        body, out_shape=jax.ShapeDtypeStruct((G, C), jnp.float32))(
            agg2, hw2p, dinv, b2, batch2d, fc_w, fc_b)


def kernel(x, edge_index, batch, W1, b1, W2, b2, fc_w, fc_b):
    pad = NW * EPW - E
    src_r = jnp.concatenate(
        [edge_index[0], jnp.zeros((pad,), jnp.int32)]).reshape(NW, CH, K)
    dst_r = jnp.concatenate(
        [edge_index[1], jnp.full((pad,), N, jnp.int32)]).reshape(NW, CH, K)

    deg16 = _make_deg()(dst_r)
    hw1 = _mm1(x, W1)
    hw1p, dinv = _scale(hw1, deg16)
    agg1 = _make_agg(H1)(hw1p, src_r, dst_r)
    hw2p = _layer2(agg1, hw1p, dinv, b1.reshape(1, H1), W2)
    agg2 = _make_agg(H2)(hw2p, src_r, dst_r)
    return _final(agg2, hw2p, dinv, b2.reshape(1, H2),
                  batch.reshape(N, 1), fc_w, fc_b)
